# Initial kernel scaffold; baseline (speedup 1.0000x reference)
import functools

import jax
import jax.numpy as jnp
from jax import lax
from jax.experimental import pallas as pl
from jax.experimental.pallas import tpu as pltpu
from jax.experimental.pallas import tpu_sc as plsc

N = 10000
D = 128
E = 160000
NPAD = 10240
EPAD = 163840
CH = 128
EPT = EPAD // 32
NCH = EPT // CH
RPT = NPAD // 16
QPR = EPAD // CH
RB = 512
GRID = NPAD // RB


def _mm_body(fu, fi, wc, wb, wf, wlu, wli, yc, yb, yf, su, si):
    x = fu[...]
    yc[...] = jnp.dot(x, wc[...], preferred_element_type=jnp.float32)
    yb[...] = jnp.dot(x, wb[...], preferred_element_type=jnp.float32)
    yf[...] = jnp.dot(x, wf[...], preferred_element_type=jnp.float32)
    su[...] = jnp.dot(x, wlu[...], preferred_element_type=jnp.float32)
    si[...] = jnp.dot(fi[...], wli[...], preferred_element_type=jnp.float32)


def _matmuls(fu, fi, wc, wb, wf, wlu, wli):
    row = pl.BlockSpec((RB, D), lambda i: (i, 0))
    wsp = pl.BlockSpec((D, D), lambda i: (0, 0))
    return pl.pallas_call(
        _mm_body,
        grid=(GRID,),
        in_specs=[row, row, wsp, wsp, wsp, wsp, wsp],
        out_specs=[row, row, row, row, row],
        out_shape=[jax.ShapeDtypeStruct((NPAD, D), jnp.float32)] * 5,
    )(fu, fi, wc, wb, wf, wlu, wli)


def _sc_scatter(yc, yb, yf, ed, zrows_c, zdeg_c, odeg_c):
    mesh = plsc.VectorSubcoreMesh(core_axis_name="c", subcore_axis_name="s")

    @functools.partial(
        pl.kernel,
        out_type=[
            jax.ShapeDtypeStruct((3, 2, NPAD, D), jnp.float32),
            jax.ShapeDtypeStruct((3, 2, NPAD, 16), jnp.float32),
        ],
        mesh=mesh,
        scratch_types=[
            pltpu.VMEM((2, CH), jnp.int32),
            pltpu.VMEM((CH, D), jnp.float32),
            pltpu.VMEM((CH, 16), jnp.float32),
            pltpu.VMEM((CH, D), jnp.float32),
            pltpu.VMEM((CH, 16), jnp.float32),
            pltpu.VMEM_SHARED((NPAD, D), jnp.float32),
            pltpu.VMEM_SHARED((NPAD, 16), jnp.float32),
            pltpu.SemaphoreType.DMA,
        ],
    )
    def body(yc_h, yb_h, yf_h, ed_h, zrows_h, zdeg_h, odeg_h,
             s_out, d_out, idx_v, rows_v, ones_v, zrows, zdeg, acc, dacc, sem):
        cid = lax.axis_index("c")
        sid = lax.axis_index("s")
        row0 = sid * RPT
        wid = cid * 16 + sid

        pltpu.sync_copy(zrows_h, zrows)
        pltpu.sync_copy(zdeg_h, zdeg)
        pltpu.sync_copy(odeg_h, ones_v)

        for r, y_h in enumerate((yc_h, yb_h, yf_h)):
            for b in range(RPT // CH):
                pltpu.sync_copy(zrows, acc.at[pl.ds(row0 + b * CH, CH)])
                pltpu.sync_copy(zdeg, dacc.at[pl.ds(row0 + b * CH, CH)])
            plsc.subcore_barrier()

            qbase = r * QPR + wid * NCH

            def ebody(i, _):
                pltpu.sync_copy(ed_h.at[qbase + i], idx_v)
                pltpu.async_copy(y_h.at[idx_v.at[0]], rows_v, sem).wait()
                pltpu.sync_copy(rows_v, acc.at[idx_v.at[1]], add=True)
                pltpu.sync_copy(ones_v, dacc.at[idx_v.at[1]], add=True)
                return 0

            lax.fori_loop(0, NCH, ebody, 0)
            plsc.subcore_barrier()

            pltpu.sync_copy(acc.at[pl.ds(row0, RPT)],
                            s_out.at[r, cid, pl.ds(row0, RPT)])
            pltpu.sync_copy(dacc.at[pl.ds(row0, RPT)],
                            d_out.at[r, cid, pl.ds(row0, RPT)])

    return body(yc, yb, yf, ed, zrows_c, zdeg_c, odeg_c)


def _comb_body(s_ref, d_ref, su_ref, si_ref, ou_ref, oi_ref):
    s = s_ref[...]
    dg = d_ref[...][..., 0]
    dc = jnp.maximum(dg[0, 0] + dg[0, 1], 1.0)
    db = jnp.maximum(dg[1, 0] + dg[1, 1], 1.0)
    df = jnp.maximum(dg[2, 0] + dg[2, 1], 1.0)
    ou_ref[...] = (s[2, 0] + s[2, 1]) / df[:, None] + su_ref[...]
    oi_ref[...] = ((s[0, 0] + s[0, 1]) / dc[:, None]
                   + (s[1, 0] + s[1, 1]) / db[:, None] + si_ref[...])


def _combine(s, dg, su, si):
    row = pl.BlockSpec((RB, D), lambda i: (i, 0))
    return pl.pallas_call(
        _comb_body,
        grid=(GRID,),
        in_specs=[
            pl.BlockSpec((3, 2, RB, D), lambda i: (0, 0, i, 0)),
            pl.BlockSpec((3, 2, RB, 16), lambda i: (0, 0, i, 0)),
            row, row,
        ],
        out_specs=[row, row],
        out_shape=[jax.ShapeDtypeStruct((NPAD, D), jnp.float32)] * 2,
    )(s, dg, su, si)


def _pack_edges(cs, cd, bs, bd, fs, fd):
    pad = EPAD - E
    ar = jnp.arange(pad, dtype=jnp.int32)
    psrc = ar % N
    pdst = N + ar % (NPAD - N)

    def one(src, dst):
        src = jnp.concatenate([src.astype(jnp.int32), psrc])
        dst = jnp.concatenate([dst.astype(jnp.int32), pdst])
        return jnp.stack([src, dst])

    ed = jnp.stack([one(cs, cd), one(bs, bd), one(fs, fd)])
    return ed.reshape(3, 2, QPR, CH).transpose(0, 2, 1, 3).reshape(3 * QPR, 2, CH)


def kernel(feat_user, feat_item, clicks_src, clicks_dst, buys_src, buys_dst,
           follows_src, follows_dst, W_clicks, W_buys, W_follows,
           W_loop_user, W_loop_item):
    fu = jnp.pad(feat_user, ((0, NPAD - N), (0, 0)))
    fi = jnp.pad(feat_item, ((0, NPAD - N), (0, 0)))
    yc, yb, yf, su, si = _matmuls(fu, fi, W_clicks, W_buys, W_follows,
                                  W_loop_user, W_loop_item)
    ed = _pack_edges(clicks_src, clicks_dst, buys_src, buys_dst,
                     follows_src, follows_dst)
    zrows_c = jnp.zeros((CH, D), jnp.float32)
    zdeg_c = jnp.zeros((CH, 16), jnp.float32)
    odeg_c = jnp.ones((CH, 16), jnp.float32)
    s, dg = _sc_scatter(yc, yb, yf, ed, zrows_c, zdeg_c, odeg_c)
    ou, oi = _combine(s, dg, su, si)
    return ou[:N], oi[:N]


# speedup vs baseline: 4.5920x; 4.5920x over previous
import functools

import jax
import jax.numpy as jnp
from jax import lax
from jax.experimental import pallas as pl
from jax.experimental.pallas import tpu as pltpu
from jax.experimental.pallas import tpu_sc as plsc

N = 10000
D = 128
E = 160000
NPAD = 10240
EPAD = 163840
CH = 128
EPT = EPAD // 32
NCH = EPT // CH
RPT = NPAD // 16
QPR = EPAD // CH
RB = 512
GRID = NPAD // RB


def _mm_body(fu, fi, wc, wb, wf, wlu, wli, yc, yb, yf, su, si):
    x = fu[...]
    yc[...] = jnp.dot(x, wc[...], preferred_element_type=jnp.float32)
    yb[...] = jnp.dot(x, wb[...], preferred_element_type=jnp.float32)
    yf[...] = jnp.dot(x, wf[...], preferred_element_type=jnp.float32)
    su[...] = jnp.dot(x, wlu[...], preferred_element_type=jnp.float32)
    si[...] = jnp.dot(fi[...], wli[...], preferred_element_type=jnp.float32)


def _matmuls(fu, fi, wc, wb, wf, wlu, wli):
    row = pl.BlockSpec((RB, D), lambda i: (i, 0))
    wsp = pl.BlockSpec((D, D), lambda i: (0, 0))
    return pl.pallas_call(
        _mm_body,
        grid=(GRID,),
        in_specs=[row, row, wsp, wsp, wsp, wsp, wsp],
        out_specs=[row, row, row, row, row],
        out_shape=[jax.ShapeDtypeStruct((NPAD, D), jnp.float32)] * 5,
    )(fu, fi, wc, wb, wf, wlu, wli)


def _sc_scatter(yc, yb, yf, ed, zrows_c, ones3_c):
    mesh = plsc.VectorSubcoreMesh(core_axis_name="c", subcore_axis_name="s")

    @functools.partial(
        pl.kernel,
        out_type=[
            jax.ShapeDtypeStruct((3, 2 * NPAD, D), jnp.float32),
            jax.ShapeDtypeStruct((2 * NPAD, D), jnp.float32),
        ],
        mesh=mesh,
        scratch_types=[
            pltpu.VMEM((2, CH), jnp.int32),
            pltpu.VMEM((CH, D), jnp.float32),
            pltpu.VMEM_SHARED((NPAD, D), jnp.float32),
            pltpu.SemaphoreType.DMA,
        ],
    )
    def body(yc_h, yb_h, yf_h, ed_h, zrows_h, ones3_h,
             s_out, d_out, idx_v, rows_v, acc, sem):
        cid = lax.axis_index("c")
        sid = lax.axis_index("s")
        row0 = sid * RPT
        wid = cid * 16 + sid

        def zero_acc():
            pltpu.sync_copy(zrows_h, rows_v)
            for b in range(RPT // CH):
                pltpu.sync_copy(rows_v, acc.at[pl.ds(row0 + b * CH, CH)])

        def flush_acc(dst_at):
            for b in range(RPT // CH):
                pltpu.sync_copy(acc.at[pl.ds(row0 + b * CH, CH)], rows_v)
                pltpu.sync_copy(
                    rows_v, dst_at(pl.ds(cid * NPAD + row0 + b * CH, CH)))

        for r, y_h in enumerate((yc_h, yb_h, yf_h)):
            zero_acc()
            plsc.subcore_barrier()
            qb = r * QPR + wid * NCH

            def ebody(i, _):
                pltpu.sync_copy(ed_h.at[qb + i], idx_v)
                pltpu.async_copy(y_h.at[idx_v.at[0]], rows_v, sem).wait()
                pltpu.sync_copy(rows_v, acc.at[idx_v.at[1]], add=True)
                return 0

            lax.fori_loop(0, NCH, ebody, 0)
            plsc.subcore_barrier()
            flush_acc(lambda ds, r=r: s_out.at[r, ds])

        zero_acc()
        plsc.subcore_barrier()
        for r in range(3):
            pltpu.sync_copy(ones3_h.at[r], rows_v)
            qb = r * QPR + wid * NCH

            def dbody(i, _):
                pltpu.sync_copy(ed_h.at[qb + i], idx_v)
                pltpu.sync_copy(rows_v, acc.at[idx_v.at[1]], add=True)
                return 0

            lax.fori_loop(0, NCH, dbody, 0)
        plsc.subcore_barrier()
        flush_acc(lambda ds: d_out.at[ds])

    return body(yc, yb, yf, ed, zrows_c, ones3_c)


def _comb_body(sc0, sc1, sb0, sb1, sf0, sf1, dg0, dg1,
               su_ref, si_ref, ou_ref, oi_ref):
    dg = dg0[0] + dg1[0]
    dc = jnp.maximum(dg[:, 0], 1.0)
    db = jnp.maximum(dg[:, 1], 1.0)
    df = jnp.maximum(dg[:, 2], 1.0)
    ou_ref[...] = (sf0[0, 0] + sf1[0, 0]) / df[:, None] + su_ref[...]
    oi_ref[...] = ((sc0[0, 0] + sc1[0, 0]) / dc[:, None]
                   + (sb0[0, 0] + sb1[0, 0]) / db[:, None] + si_ref[...])


def _combine(s, dg, su, si):
    row = pl.BlockSpec((RB, D), lambda i: (i, 0))

    def sspec(r, c):
        return pl.BlockSpec((1, 1, RB, D), lambda i, r=r, c=c: (r, c, i, 0))

    def dspec(c):
        return pl.BlockSpec((1, RB, D), lambda i, c=c: (c, i, 0))

    return pl.pallas_call(
        _comb_body,
        grid=(GRID,),
        in_specs=[
            sspec(0, 0), sspec(0, 1), sspec(1, 0), sspec(1, 1),
            sspec(2, 0), sspec(2, 1),
            dspec(0), dspec(1),
            row, row,
        ],
        out_specs=[row, row],
        out_shape=[jax.ShapeDtypeStruct((NPAD, D), jnp.float32)] * 2,
    )(s, s, s, s, s, s, dg, dg, su, si)


def _pack_edges(cs, cd, bs, bd, fs, fd):
    pad = EPAD - E
    ar = jnp.arange(pad, dtype=jnp.int32)
    psrc = ar % N
    pdst = N + ar % (NPAD - N)

    def one(src, dst):
        src = jnp.concatenate([src.astype(jnp.int32), psrc])
        dst = jnp.concatenate([dst.astype(jnp.int32), pdst])
        return jnp.stack([src, dst])

    ed = jnp.stack([one(cs, cd), one(bs, bd), one(fs, fd)])
    return ed.reshape(3, 2, QPR, CH).transpose(0, 2, 1, 3).reshape(3 * QPR, 2, CH)


def kernel(feat_user, feat_item, clicks_src, clicks_dst, buys_src, buys_dst,
           follows_src, follows_dst, W_clicks, W_buys, W_follows,
           W_loop_user, W_loop_item):
    fu = jnp.pad(feat_user, ((0, NPAD - N), (0, 0)))
    fi = jnp.pad(feat_item, ((0, NPAD - N), (0, 0)))
    yc, yb, yf, su, si = _matmuls(fu, fi, W_clicks, W_buys, W_follows,
                                  W_loop_user, W_loop_item)
    ed = _pack_edges(clicks_src, clicks_dst, buys_src, buys_dst,
                     follows_src, follows_dst)
    zrows_c = jnp.zeros((CH, D), jnp.float32)
    ones3_c = jnp.broadcast_to(
        (jnp.arange(D)[None, None, :] == jnp.arange(3)[:, None, None])
        .astype(jnp.float32), (3, CH, D))
    s, dg = _sc_scatter(yc, yb, yf, ed, zrows_c, ones3_c)
    s = s.reshape(3, 2, NPAD, D)
    dg = dg.reshape(2, NPAD, D)
    ou, oi = _combine(s, dg, su, si)
    return ou[:N], oi[:N]


# speedup vs baseline: 4.7490x; 1.0342x over previous
import functools

import jax
import jax.numpy as jnp
from jax import lax
from jax.experimental import pallas as pl
from jax.experimental.pallas import tpu as pltpu
from jax.experimental.pallas import tpu_sc as plsc

N = 10000
D = 128
E = 160000
NPAD = 10240
EPAD = 163840
CH = 80
EPT = EPAD // 32
NCH = EPT // CH
RPT = NPAD // 16
QPR = EPAD // CH
EDQ = 3 * QPR + 8
RB = 512
GRID = NPAD // RB


def _mm_body(fu, fi, wc, wb, wf, wlu, wli, yc, yb, yf, su, si):
    x = fu[...]
    yc[...] = jnp.dot(x, wc[...], preferred_element_type=jnp.float32)
    yb[...] = jnp.dot(x, wb[...], preferred_element_type=jnp.float32)
    yf[...] = jnp.dot(x, wf[...], preferred_element_type=jnp.float32)
    su[...] = jnp.dot(x, wlu[...], preferred_element_type=jnp.float32)
    si[...] = jnp.dot(fi[...], wli[...], preferred_element_type=jnp.float32)


def _matmuls(fu, fi, wc, wb, wf, wlu, wli):
    row = pl.BlockSpec((RB, D), lambda i: (i, 0))
    wsp = pl.BlockSpec((D, D), lambda i: (0, 0))
    return pl.pallas_call(
        _mm_body,
        grid=(GRID,),
        in_specs=[row, row, wsp, wsp, wsp, wsp, wsp],
        out_specs=[row, row, row, row, row],
        out_shape=[jax.ShapeDtypeStruct((NPAD, D), jnp.float32)] * 5,
    )(fu, fi, wc, wb, wf, wlu, wli)


def _sc_scatter(yc, yb, yf, ed, zrows_c, ones3_c):
    mesh = plsc.VectorSubcoreMesh(core_axis_name="c", subcore_axis_name="s")

    @functools.partial(
        pl.kernel,
        out_type=[
            jax.ShapeDtypeStruct((3, 2 * NPAD, D), jnp.float32),
            jax.ShapeDtypeStruct((2 * NPAD, D), jnp.float32),
        ],
        mesh=mesh,
        scratch_types=[
            pltpu.VMEM((4, 2, CH), jnp.int32),
            pltpu.VMEM((2, CH, D), jnp.float32),
            pltpu.VMEM_SHARED((NPAD, D), jnp.float32),
            pltpu.SemaphoreType.DMA,
            pltpu.SemaphoreType.DMA,
            pltpu.SemaphoreType.DMA,
            pltpu.SemaphoreType.DMA,
        ],
    )
    def body(yc_h, yb_h, yf_h, ed_h, zrows_h, ones3_h,
             s_out, d_out, idx4, rows2, acc, gsem0, gsem1, ssem0, ssem1):
        cid = lax.axis_index("c")
        sid = lax.axis_index("s")
        row0 = sid * RPT
        wid = cid * 16 + sid
        gsem = (gsem0, gsem1)
        ssem = (ssem0, ssem1)

        def zero_acc():
            pltpu.sync_copy(zrows_h, rows2.at[0])
            for b in range(RPT // CH):
                pltpu.sync_copy(rows2.at[0], acc.at[pl.ds(row0 + b * CH, CH)])

        def flush_acc(dst_at):
            for b in range(RPT // CH):
                pltpu.sync_copy(acc.at[pl.ds(row0 + b * CH, CH)], rows2.at[0])
                pltpu.sync_copy(
                    rows2.at[0], dst_at(pl.ds(cid * NPAD + row0 + b * CH, CH)))

        def g_start(y_h, k, p):
            return pltpu.async_copy(
                y_h.at[idx4.at[k, 0]], rows2.at[p], gsem[p])

        def g_wait(y_h, k, p):
            pltpu.make_async_copy(
                y_h.at[idx4.at[k, 0]], rows2.at[p], gsem[p]).wait()

        def s_sync(k, p):
            pltpu.sync_copy(rows2.at[p], acc.at[idx4.at[k, 1]], add=True)

        for r, y_h in enumerate((yc_h, yb_h, yf_h)):
            zero_acc()
            plsc.subcore_barrier()
            qb = r * QPR + wid * NCH

            pltpu.sync_copy(ed_h.at[qb], idx4.at[0])
            g_start(y_h, 0, 0)

            def pair(j, _):
                c = qb + 2 * j
                pltpu.sync_copy(ed_h.at[c + 1], idx4.at[1])
                g_start(y_h, 1, 1)
                g_wait(y_h, 0, 0)
                s_sync(0, 0)
                pltpu.sync_copy(ed_h.at[c + 2], idx4.at[0])
                g_start(y_h, 0, 0)
                g_wait(y_h, 1, 1)
                s_sync(1, 1)
                return 0

            lax.fori_loop(0, NCH // 2, pair, 0)
            g_wait(y_h, 0, 0)
            plsc.subcore_barrier()
            flush_acc(lambda ds, r=r: s_out.at[r, ds])

        zero_acc()
        plsc.subcore_barrier()
        for r in range(3):
            pltpu.sync_copy(ones3_h.at[r], rows2.at[0])
            qb = r * QPR + wid * NCH

            def dstep(i, _):
                pltpu.sync_copy(ed_h.at[qb + i], idx4.at[0])
                s_sync(0, 0)
                return 0

            lax.fori_loop(0, NCH, dstep, 0)
        plsc.subcore_barrier()
        flush_acc(lambda ds: d_out.at[ds])

    return body(yc, yb, yf, ed, zrows_c, ones3_c)


def _comb_body(sc0, sc1, sb0, sb1, sf0, sf1, dg0, dg1,
               su_ref, si_ref, ou_ref, oi_ref):
    dg = dg0[0] + dg1[0]
    dc = jnp.maximum(dg[:, 0], 1.0)
    db = jnp.maximum(dg[:, 1], 1.0)
    df = jnp.maximum(dg[:, 2], 1.0)
    ou_ref[...] = (sf0[0, 0] + sf1[0, 0]) / df[:, None] + su_ref[...]
    oi_ref[...] = ((sc0[0, 0] + sc1[0, 0]) / dc[:, None]
                   + (sb0[0, 0] + sb1[0, 0]) / db[:, None] + si_ref[...])


def _combine(s, dg, su, si):
    row = pl.BlockSpec((RB, D), lambda i: (i, 0))

    def sspec(r, c):
        return pl.BlockSpec((1, 1, RB, D), lambda i, r=r, c=c: (r, c, i, 0))

    def dspec(c):
        return pl.BlockSpec((1, RB, D), lambda i, c=c: (c, i, 0))

    return pl.pallas_call(
        _comb_body,
        grid=(GRID,),
        in_specs=[
            sspec(0, 0), sspec(0, 1), sspec(1, 0), sspec(1, 1),
            sspec(2, 0), sspec(2, 1),
            dspec(0), dspec(1),
            row, row,
        ],
        out_specs=[row, row],
        out_shape=[jax.ShapeDtypeStruct((NPAD, D), jnp.float32)] * 2,
    )(s, s, s, s, s, s, dg, dg, su, si)


def _pack_edges(cs, cd, bs, bd, fs, fd):
    pad = EPAD - E
    ar = jnp.arange(pad, dtype=jnp.int32)
    psrc = ar % N
    pdst = N + ar % (NPAD - N)

    def one(src, dst):
        src = jnp.concatenate([src.astype(jnp.int32), psrc])
        dst = jnp.concatenate([dst.astype(jnp.int32), pdst])
        return jnp.stack([src, dst])

    ed = jnp.stack([one(cs, cd), one(bs, bd), one(fs, fd)])
    ed = ed.reshape(3, 2, QPR, CH).transpose(0, 2, 1, 3).reshape(3 * QPR, 2, CH)
    return jnp.pad(ed, ((0, EDQ - 3 * QPR), (0, 0), (0, 0)))


def kernel(feat_user, feat_item, clicks_src, clicks_dst, buys_src, buys_dst,
           follows_src, follows_dst, W_clicks, W_buys, W_follows,
           W_loop_user, W_loop_item):
    fu = jnp.pad(feat_user, ((0, NPAD - N), (0, 0)))
    fi = jnp.pad(feat_item, ((0, NPAD - N), (0, 0)))
    yc, yb, yf, su, si = _matmuls(fu, fi, W_clicks, W_buys, W_follows,
                                  W_loop_user, W_loop_item)
    ed = _pack_edges(clicks_src, clicks_dst, buys_src, buys_dst,
                     follows_src, follows_dst)
    zrows_c = jnp.zeros((CH, D), jnp.float32)
    ones3_c = jnp.broadcast_to(
        (jnp.arange(D)[None, None, :] == jnp.arange(3)[:, None, None])
        .astype(jnp.float32), (3, CH, D))
    s, dg = _sc_scatter(yc, yb, yf, ed, zrows_c, ones3_c)
    s = s.reshape(3, 2, NPAD, D)
    dg = dg.reshape(2, NPAD, D)
    ou, oi = _combine(s, dg, su, si)
    return ou[:N], oi[:N]


# speedup vs baseline: 5.4701x; 1.1518x over previous
import functools

import jax
import jax.numpy as jnp
from jax import lax
from jax.experimental import pallas as pl
from jax.experimental.pallas import tpu as pltpu
from jax.experimental.pallas import tpu_sc as plsc

N = 10000
D = 128
E = 160000
NPAD = 10240
EPAD = 163840
CH = 80
EPT = EPAD // 32
NCH = EPT // CH
RPT = NPAD // 16
QPR = EPAD // CH
EDQ = 3 * QPR + 8
RB = 512
GRID = NPAD // RB


def _mm_body(fu, fi, wc, wb, wf, wlu, wli, yc, yb, yf, su, si):
    x = fu[...]
    yc[...] = jnp.dot(x, wc[...], preferred_element_type=jnp.float32)
    yb[...] = jnp.dot(x, wb[...], preferred_element_type=jnp.float32)
    yf[...] = jnp.dot(x, wf[...], preferred_element_type=jnp.float32)
    su[...] = jnp.dot(x, wlu[...], preferred_element_type=jnp.float32)
    si[...] = jnp.dot(fi[...], wli[...], preferred_element_type=jnp.float32)


def _matmuls(fu, fi, wc, wb, wf, wlu, wli):
    row = pl.BlockSpec((RB, D), lambda i: (i, 0))
    wsp = pl.BlockSpec((D, D), lambda i: (0, 0))
    return pl.pallas_call(
        _mm_body,
        grid=(GRID,),
        in_specs=[row, row, wsp, wsp, wsp, wsp, wsp],
        out_specs=[row, row, row, row, row],
        out_shape=[jax.ShapeDtypeStruct((NPAD, D), jnp.float32)] * 5,
    )(fu, fi, wc, wb, wf, wlu, wli)


def _sc_scatter(yc, yb, yf, ed, dd, zrows_c, ones3_c):
    mesh = plsc.VectorSubcoreMesh(core_axis_name="c", subcore_axis_name="s")

    @functools.partial(
        pl.kernel,
        out_type=[
            jax.ShapeDtypeStruct((3, 2 * NPAD, D), jnp.float32),
            jax.ShapeDtypeStruct((2 * NPAD, D), jnp.float32),
        ],
        mesh=mesh,
        scratch_types=[
            pltpu.VMEM((4, 2, CH), jnp.int32),
            pltpu.VMEM((8, 128), jnp.int32),
            pltpu.VMEM((2 * CH, D), jnp.float32),
            pltpu.VMEM_SHARED((NPAD, D), jnp.float32),
            pltpu.SemaphoreType.DMA,
            pltpu.SemaphoreType.DMA,
        ],
    )
    def body(yc_h, yb_h, yf_h, ed_h, dd_h, zrows_h, ones3_h,
             s_out, d_out, idx4, didx, rowsbuf, acc, gsem0, gsem1):
        cid = lax.axis_index("c")
        sid = lax.axis_index("s")
        row0 = sid * RPT
        wid = cid * 16 + sid
        gsem = (gsem0, gsem1)
        rbuf = (rowsbuf.at[pl.ds(0, CH)], rowsbuf.at[pl.ds(CH, CH)])
        dbuf = rowsbuf.at[pl.ds(0, 128)]

        def zero_acc():
            pltpu.sync_copy(zrows_h, dbuf)
            for b in range(RPT // 128):
                pltpu.sync_copy(dbuf, acc.at[pl.ds(row0 + b * 128, 128)])

        def flush_acc(dst_at):
            for b in range(RPT // 128):
                pltpu.sync_copy(acc.at[pl.ds(row0 + b * 128, 128)], dbuf)
                pltpu.sync_copy(
                    dbuf, dst_at(pl.ds(cid * NPAD + row0 + b * 128, 128)))

        def g_start(y_h, k, p):
            return pltpu.async_copy(
                y_h.at[idx4.at[k, 0]], rbuf[p], gsem[p])

        def g_wait(y_h, k, p):
            pltpu.make_async_copy(
                y_h.at[idx4.at[k, 0]], rbuf[p], gsem[p]).wait()

        def s_sync(k, p):
            pltpu.sync_copy(rbuf[p], acc.at[idx4.at[k, 1]], add=True)

        for r, y_h in enumerate((yc_h, yb_h, yf_h)):
            zero_acc()
            plsc.subcore_barrier()
            qb = r * QPR + wid * NCH

            pltpu.sync_copy(ed_h.at[qb], idx4.at[0])
            g_start(y_h, 0, 0)

            def pair(j, _):
                c = qb + 2 * j
                pltpu.sync_copy(ed_h.at[c + 1], idx4.at[1])
                g_start(y_h, 1, 1)
                g_wait(y_h, 0, 0)
                s_sync(0, 0)
                pltpu.sync_copy(ed_h.at[c + 2], idx4.at[0])
                g_start(y_h, 0, 0)
                g_wait(y_h, 1, 1)
                s_sync(1, 1)
                return 0

            lax.fori_loop(0, NCH // 2, pair, 0)
            g_wait(y_h, 0, 0)
            plsc.subcore_barrier()
            flush_acc(lambda ds, r=r: s_out.at[r, ds])

        zero_acc()
        plsc.subcore_barrier()
        for r in range(3):
            pltpu.sync_copy(ones3_h.at[r], dbuf)
            qb2 = (r * EPAD + wid * EPT) // 128

            def dbatch(t, _):
                pltpu.sync_copy(dd_h.at[pl.ds(pl.multiple_of(qb2 + 8 * t, 8), 8)], didx)
                for k in range(8):
                    pltpu.sync_copy(dbuf, acc.at[didx.at[k]], add=True)
                return 0

            lax.fori_loop(0, EPT // 1024, dbatch, 0)
        plsc.subcore_barrier()
        flush_acc(lambda ds: d_out.at[ds])

    return body(yc, yb, yf, ed, dd, zrows_c, ones3_c)


def _comb_body(sc0, sc1, sb0, sb1, sf0, sf1, dg0, dg1,
               su_ref, si_ref, ou_ref, oi_ref):
    dg = dg0[0] + dg1[0]
    dc = jnp.maximum(dg[:, 0], 1.0)
    db = jnp.maximum(dg[:, 1], 1.0)
    df = jnp.maximum(dg[:, 2], 1.0)
    ou_ref[...] = (sf0[0, 0] + sf1[0, 0]) / df[:, None] + su_ref[...]
    oi_ref[...] = ((sc0[0, 0] + sc1[0, 0]) / dc[:, None]
                   + (sb0[0, 0] + sb1[0, 0]) / db[:, None] + si_ref[...])


def _combine(s, dg, su, si):
    row = pl.BlockSpec((RB, D), lambda i: (i, 0))

    def sspec(r, c):
        return pl.BlockSpec((1, 1, RB, D), lambda i, r=r, c=c: (r, c, i, 0))

    def dspec(c):
        return pl.BlockSpec((1, RB, D), lambda i, c=c: (c, i, 0))

    return pl.pallas_call(
        _comb_body,
        grid=(GRID,),
        in_specs=[
            sspec(0, 0), sspec(0, 1), sspec(1, 0), sspec(1, 1),
            sspec(2, 0), sspec(2, 1),
            dspec(0), dspec(1),
            row, row,
        ],
        out_specs=[row, row],
        out_shape=[jax.ShapeDtypeStruct((NPAD, D), jnp.float32)] * 2,
    )(s, s, s, s, s, s, dg, dg, su, si)


def _pack_edges(cs, cd, bs, bd, fs, fd):
    pad = EPAD - E
    ar = jnp.arange(pad, dtype=jnp.int32)
    psrc = ar % N
    pdst = N + ar % (NPAD - N)

    def one(src, dst):
        src = jnp.concatenate([src.astype(jnp.int32), psrc])
        dst = jnp.concatenate([dst.astype(jnp.int32), pdst])
        return jnp.stack([src, dst])

    ed = jnp.stack([one(cs, cd), one(bs, bd), one(fs, fd)])
    dd = ed[:, 1].reshape(3 * EPAD // 128, 128)
    ed = ed.reshape(3, 2, QPR, CH).transpose(0, 2, 1, 3).reshape(3 * QPR, 2, CH)
    return jnp.pad(ed, ((0, EDQ - 3 * QPR), (0, 0), (0, 0))), dd


def kernel(feat_user, feat_item, clicks_src, clicks_dst, buys_src, buys_dst,
           follows_src, follows_dst, W_clicks, W_buys, W_follows,
           W_loop_user, W_loop_item):
    fu = jnp.pad(feat_user, ((0, NPAD - N), (0, 0)))
    fi = jnp.pad(feat_item, ((0, NPAD - N), (0, 0)))
    yc, yb, yf, su, si = _matmuls(fu, fi, W_clicks, W_buys, W_follows,
                                  W_loop_user, W_loop_item)
    ed, dd = _pack_edges(clicks_src, clicks_dst, buys_src, buys_dst,
                         follows_src, follows_dst)
    zrows_c = jnp.zeros((128, D), jnp.float32)
    ones3_c = jnp.broadcast_to(
        (jnp.arange(D)[None, None, :] == jnp.arange(3)[:, None, None])
        .astype(jnp.float32), (3, 128, D))
    s, dg = _sc_scatter(yc, yb, yf, ed, dd, zrows_c, ones3_c)
    s = s.reshape(3, 2, NPAD, D)
    dg = dg.reshape(2, NPAD, D)
    ou, oi = _combine(s, dg, su, si)
    return ou[:N], oi[:N]


# speedup vs baseline: 5.6531x; 1.0335x over previous
import functools

import jax
import jax.numpy as jnp
from jax import lax
from jax.experimental import pallas as pl
from jax.experimental.pallas import tpu as pltpu
from jax.experimental.pallas import tpu_sc as plsc

N = 10000
D = 128
E = 160000
NPAD = 10240
EPAD = 163840
CH = 80
EPT = EPAD // 32
NCH = EPT // CH
RPT = NPAD // 16
QPR = EPAD // CH
EDQ = 3 * QPR + 8
RB = 512
GRID = NPAD // RB


def _mm_body(fu, fi, wc, wb, wf, wlu, wli, yc, yb, yf, su, si):
    x = fu[...]
    yc[...] = jnp.dot(x, wc[...], preferred_element_type=jnp.float32)
    yb[...] = jnp.dot(x, wb[...], preferred_element_type=jnp.float32)
    yf[...] = jnp.dot(x, wf[...], preferred_element_type=jnp.float32)
    su[...] = jnp.dot(x, wlu[...], preferred_element_type=jnp.float32)
    si[...] = jnp.dot(fi[...], wli[...], preferred_element_type=jnp.float32)


def _matmuls(fu, fi, wc, wb, wf, wlu, wli):
    row = pl.BlockSpec((RB, D), lambda i: (i, 0))
    wsp = pl.BlockSpec((D, D), lambda i: (0, 0))
    return pl.pallas_call(
        _mm_body,
        grid=(GRID,),
        in_specs=[row, row, wsp, wsp, wsp, wsp, wsp],
        out_specs=[row, row, row, row, row],
        out_shape=[jax.ShapeDtypeStruct((NPAD, D), jnp.float32)] * 5,
    )(fu, fi, wc, wb, wf, wlu, wli)


def _sc_scatter(yc, yb, yf, ed, dd, zrows_c, ones3_c):
    mesh = plsc.VectorSubcoreMesh(core_axis_name="c", subcore_axis_name="s")

    @functools.partial(
        pl.kernel,
        out_type=[
            jax.ShapeDtypeStruct((3, 2 * NPAD, D), jnp.float32),
            jax.ShapeDtypeStruct((2 * NPAD, D), jnp.float32),
        ],
        mesh=mesh,
        scratch_types=[
            pltpu.VMEM((8, 2, CH), jnp.int32),
            pltpu.VMEM((8, 128), jnp.int32),
            pltpu.VMEM((2 * CH, D), jnp.float32),
            pltpu.VMEM_SHARED((NPAD, D), jnp.float32),
            pltpu.SemaphoreType.DMA,
            pltpu.SemaphoreType.DMA,
        ],
    )
    def body(yc_h, yb_h, yf_h, ed_h, dd_h, zrows_h, ones3_h,
             s_out, d_out, idx8, didx, rowsbuf, acc, gsem0, gsem1):
        cid = lax.axis_index("c")
        sid = lax.axis_index("s")
        row0 = sid * RPT
        wid = cid * 16 + sid
        gsem = (gsem0, gsem1)
        rbuf = (rowsbuf.at[pl.ds(0, CH)], rowsbuf.at[pl.ds(CH, CH)])
        dbuf = rowsbuf.at[pl.ds(0, 128)]

        def zero_acc():
            pltpu.sync_copy(zrows_h, dbuf)
            for b in range(RPT // 128):
                pltpu.sync_copy(dbuf, acc.at[pl.ds(row0 + b * 128, 128)])

        def flush_acc(dst_at):
            for b in range(RPT // 128):
                pltpu.sync_copy(acc.at[pl.ds(row0 + b * 128, 128)], dbuf)
                pltpu.sync_copy(
                    dbuf, dst_at(pl.ds(cid * NPAD + row0 + b * 128, 128)))

        def g_start(y_h, k, p):
            return pltpu.async_copy(
                y_h.at[idx8.at[k, 0]], rbuf[p], gsem[p])

        def g_wait(y_h, k, p):
            pltpu.make_async_copy(
                y_h.at[idx8.at[k, 0]], rbuf[p], gsem[p]).wait()

        def s_sync(k, p):
            pltpu.sync_copy(rbuf[p], acc.at[idx8.at[k, 1]], add=True)

        for r, y_h in enumerate((yc_h, yb_h, yf_h)):
            zero_acc()
            plsc.subcore_barrier()
            qb = r * QPR + wid * NCH

            def fbatch(t, _):
                c0 = pl.multiple_of(qb + 8 * t, 8)
                pltpu.sync_copy(ed_h.at[pl.ds(c0, 8)], idx8)
                g_start(y_h, 0, 0)
                for k in range(7):
                    g_start(y_h, k + 1, (k + 1) % 2)
                    g_wait(y_h, k, k % 2)
                    s_sync(k, k % 2)
                g_wait(y_h, 7, 1)
                s_sync(7, 1)
                return 0

            lax.fori_loop(0, NCH // 8, fbatch, 0)
            plsc.subcore_barrier()
            flush_acc(lambda ds, r=r: s_out.at[r, ds])

        zero_acc()
        plsc.subcore_barrier()
        for r in range(3):
            pltpu.sync_copy(ones3_h.at[r], dbuf)
            qb2 = (r * EPAD + wid * EPT) // 128

            def dbatch(t, _):
                pltpu.sync_copy(dd_h.at[pl.ds(pl.multiple_of(qb2 + 8 * t, 8), 8)], didx)
                for k in range(8):
                    pltpu.sync_copy(dbuf, acc.at[didx.at[k]], add=True)
                return 0

            lax.fori_loop(0, EPT // 1024, dbatch, 0)
        plsc.subcore_barrier()
        flush_acc(lambda ds: d_out.at[ds])

    return body(yc, yb, yf, ed, dd, zrows_c, ones3_c)


def _comb_body(sc0, sc1, sb0, sb1, sf0, sf1, dg0, dg1,
               su_ref, si_ref, ou_ref, oi_ref):
    dg = dg0[0] + dg1[0]
    dc = jnp.maximum(dg[:, 0], 1.0)
    db = jnp.maximum(dg[:, 1], 1.0)
    df = jnp.maximum(dg[:, 2], 1.0)
    ou_ref[...] = (sf0[0, 0] + sf1[0, 0]) / df[:, None] + su_ref[...]
    oi_ref[...] = ((sc0[0, 0] + sc1[0, 0]) / dc[:, None]
                   + (sb0[0, 0] + sb1[0, 0]) / db[:, None] + si_ref[...])


def _combine(s, dg, su, si):
    row = pl.BlockSpec((RB, D), lambda i: (i, 0))

    def sspec(r, c):
        return pl.BlockSpec((1, 1, RB, D), lambda i, r=r, c=c: (r, c, i, 0))

    def dspec(c):
        return pl.BlockSpec((1, RB, D), lambda i, c=c: (c, i, 0))

    return pl.pallas_call(
        _comb_body,
        grid=(GRID,),
        in_specs=[
            sspec(0, 0), sspec(0, 1), sspec(1, 0), sspec(1, 1),
            sspec(2, 0), sspec(2, 1),
            dspec(0), dspec(1),
            row, row,
        ],
        out_specs=[row, row],
        out_shape=[jax.ShapeDtypeStruct((NPAD, D), jnp.float32)] * 2,
    )(s, s, s, s, s, s, dg, dg, su, si)


def _pack_edges(cs, cd, bs, bd, fs, fd):
    pad = EPAD - E
    ar = jnp.arange(pad, dtype=jnp.int32)
    psrc = ar % N
    pdst = N + ar % (NPAD - N)

    def one(src, dst):
        src = jnp.concatenate([src.astype(jnp.int32), psrc])
        dst = jnp.concatenate([dst.astype(jnp.int32), pdst])
        return jnp.stack([src, dst])

    ed = jnp.stack([one(cs, cd), one(bs, bd), one(fs, fd)])
    dd = ed[:, 1].reshape(3 * EPAD // 128, 128)
    ed = ed.reshape(3, 2, QPR, CH).transpose(0, 2, 1, 3).reshape(3 * QPR, 2, CH)
    return jnp.pad(ed, ((0, EDQ - 3 * QPR), (0, 0), (0, 0))), dd


def kernel(feat_user, feat_item, clicks_src, clicks_dst, buys_src, buys_dst,
           follows_src, follows_dst, W_clicks, W_buys, W_follows,
           W_loop_user, W_loop_item):
    fu = jnp.pad(feat_user, ((0, NPAD - N), (0, 0)))
    fi = jnp.pad(feat_item, ((0, NPAD - N), (0, 0)))
    yc, yb, yf, su, si = _matmuls(fu, fi, W_clicks, W_buys, W_follows,
                                  W_loop_user, W_loop_item)
    ed, dd = _pack_edges(clicks_src, clicks_dst, buys_src, buys_dst,
                         follows_src, follows_dst)
    zrows_c = jnp.zeros((128, D), jnp.float32)
    ones3_c = jnp.broadcast_to(
        (jnp.arange(D)[None, None, :] == jnp.arange(3)[:, None, None])
        .astype(jnp.float32), (3, 128, D))
    s, dg = _sc_scatter(yc, yb, yf, ed, dd, zrows_c, ones3_c)
    s = s.reshape(3, 2, NPAD, D)
    dg = dg.reshape(2, NPAD, D)
    ou, oi = _combine(s, dg, su, si)
    return ou[:N], oi[:N]


# speedup vs baseline: 5.6778x; 1.0044x over previous
import functools

import jax
import jax.numpy as jnp
from jax import lax
from jax.experimental import pallas as pl
from jax.experimental.pallas import tpu as pltpu
from jax.experimental.pallas import tpu_sc as plsc

N = 10000
D = 128
E = 160000
NPAD = 10240
EPAD = 163840
CH = 80
EPT = EPAD // 32
NCH = EPT // CH
RPT = NPAD // 16
QPR = EPAD // CH
EDQ = 3 * QPR + 8
RB = 512
GRID = NPAD // RB


def _mm_body(fu, fi, wc, wb, wf, wlu, wli, yc, yb, yf, su, si):
    x = fu[...]
    yc[...] = jnp.dot(x, wc[...], preferred_element_type=jnp.float32)
    yb[...] = jnp.dot(x, wb[...], preferred_element_type=jnp.float32)
    yf[...] = jnp.dot(x, wf[...], preferred_element_type=jnp.float32)
    su[...] = jnp.dot(x, wlu[...], preferred_element_type=jnp.float32)
    si[...] = jnp.dot(fi[...], wli[...], preferred_element_type=jnp.float32)


def _matmuls(fu, fi, wc, wb, wf, wlu, wli):
    row = pl.BlockSpec((RB, D), lambda i: (i, 0))
    wsp = pl.BlockSpec((D, D), lambda i: (0, 0))
    return pl.pallas_call(
        _mm_body,
        grid=(GRID,),
        in_specs=[row, row, wsp, wsp, wsp, wsp, wsp],
        out_specs=[row, row, row, row, row],
        out_shape=[jax.ShapeDtypeStruct((NPAD, D), jnp.float32)] * 5,
    )(fu, fi, wc, wb, wf, wlu, wli)


def _sc_scatter(yc, yb, yf, ed, dd, zrows_c, ones3_c):
    mesh = plsc.VectorSubcoreMesh(core_axis_name="c", subcore_axis_name="s")

    @functools.partial(
        pl.kernel,
        out_type=[
            jax.ShapeDtypeStruct((3, 2 * NPAD, D), jnp.float32),
            jax.ShapeDtypeStruct((2 * NPAD, D), jnp.float32),
        ],
        mesh=mesh,
        scratch_types=[
            pltpu.VMEM((8, 2, CH), jnp.int32),
            pltpu.VMEM((8, 128), jnp.int32),
            pltpu.VMEM((2 * CH, D), jnp.float32),
            pltpu.VMEM_SHARED((NPAD, D), jnp.float32),
            pltpu.SemaphoreType.DMA,
            pltpu.SemaphoreType.DMA,
        ],
    )
    def body(yc_h, yb_h, yf_h, ed_h, dd_h, zrows_h, ones3_h,
             s_out, d_out, idx8, didx, rowsbuf, acc, gsem0, gsem1):
        cid = lax.axis_index("c")
        sid = lax.axis_index("s")
        row0 = sid * RPT
        wid = cid * 16 + sid
        gsem = (gsem0, gsem1)
        rbuf = (rowsbuf.at[pl.ds(0, CH)], rowsbuf.at[pl.ds(CH, CH)])
        dbuf = rowsbuf.at[pl.ds(0, 128)]

        def zero_acc():
            pltpu.sync_copy(zrows_h, dbuf)
            for b in range(RPT // 128):
                pltpu.sync_copy(dbuf, acc.at[pl.ds(row0 + b * 128, 128)])

        def flush_acc(dst_at):
            for b in range(RPT // 128):
                pltpu.sync_copy(acc.at[pl.ds(row0 + b * 128, 128)], dbuf)
                pltpu.sync_copy(
                    dbuf, dst_at(pl.ds(cid * NPAD + row0 + b * 128, 128)))

        def g_start(y_h, k, p):
            return pltpu.async_copy(
                y_h.at[idx8.at[k, 0]], rbuf[p], gsem[p])

        def g_wait(y_h, k, p):
            pltpu.make_async_copy(
                y_h.at[idx8.at[k, 0]], rbuf[p], gsem[p]).wait()

        def s_sync(k, p):
            pltpu.sync_copy(rbuf[p], acc.at[idx8.at[k, 1]], add=True)

        for r, y_h in enumerate((yc_h, yb_h, yf_h)):
            zero_acc()
            plsc.subcore_barrier()
            qb = r * QPR + wid * NCH

            def fbatch(t, _):
                c0 = pl.multiple_of(qb + 8 * t, 8)
                pltpu.sync_copy(ed_h.at[pl.ds(c0, 8)], idx8)
                g_start(y_h, 0, 0)
                for k in range(7):
                    g_start(y_h, k + 1, (k + 1) % 2)
                    g_wait(y_h, k, k % 2)
                    s_sync(k, k % 2)
                g_wait(y_h, 7, 1)
                s_sync(7, 1)
                return 0

            lax.fori_loop(0, NCH // 8, fbatch, 0)
            plsc.subcore_barrier()
            flush_acc(lambda ds, r=r: s_out.at[r, ds])

        zero_acc()
        plsc.subcore_barrier()
        for r in range(3):
            pltpu.sync_copy(ones3_h.at[r], dbuf)
            qb2 = (r * EPAD + wid * EPT) // 128

            def dbatch(t, _):
                pltpu.sync_copy(dd_h.at[pl.ds(pl.multiple_of(qb2 + 8 * t, 8), 8)], didx)
                for k in range(8):
                    pltpu.async_copy(dbuf, acc.at[didx.at[k]], gsem0, add=True)
                for k in range(8):
                    pltpu.make_async_copy(dbuf, acc.at[didx.at[k]], gsem0).wait()
                return 0

            lax.fori_loop(0, EPT // 1024, dbatch, 0)
        plsc.subcore_barrier()
        flush_acc(lambda ds: d_out.at[ds])

    return body(yc, yb, yf, ed, dd, zrows_c, ones3_c)


def _comb_body(sc0, sc1, sb0, sb1, sf0, sf1, dg0, dg1,
               su_ref, si_ref, ou_ref, oi_ref):
    dg = dg0[0] + dg1[0]
    dc = jnp.maximum(dg[:, 0], 1.0)
    db = jnp.maximum(dg[:, 1], 1.0)
    df = jnp.maximum(dg[:, 2], 1.0)
    ou_ref[...] = (sf0[0, 0] + sf1[0, 0]) / df[:, None] + su_ref[...]
    oi_ref[...] = ((sc0[0, 0] + sc1[0, 0]) / dc[:, None]
                   + (sb0[0, 0] + sb1[0, 0]) / db[:, None] + si_ref[...])


def _combine(s, dg, su, si):
    row = pl.BlockSpec((RB, D), lambda i: (i, 0))

    def sspec(r, c):
        return pl.BlockSpec((1, 1, RB, D), lambda i, r=r, c=c: (r, c, i, 0))

    def dspec(c):
        return pl.BlockSpec((1, RB, D), lambda i, c=c: (c, i, 0))

    return pl.pallas_call(
        _comb_body,
        grid=(GRID,),
        in_specs=[
            sspec(0, 0), sspec(0, 1), sspec(1, 0), sspec(1, 1),
            sspec(2, 0), sspec(2, 1),
            dspec(0), dspec(1),
            row, row,
        ],
        out_specs=[row, row],
        out_shape=[jax.ShapeDtypeStruct((NPAD, D), jnp.float32)] * 2,
    )(s, s, s, s, s, s, dg, dg, su, si)


def _pack_edges(cs, cd, bs, bd, fs, fd):
    pad = EPAD - E
    ar = jnp.arange(pad, dtype=jnp.int32)
    psrc = ar % N
    pdst = N + ar % (NPAD - N)

    def one(src, dst):
        src = jnp.concatenate([src.astype(jnp.int32), psrc])
        dst = jnp.concatenate([dst.astype(jnp.int32), pdst])
        return jnp.stack([src, dst])

    ed = jnp.stack([one(cs, cd), one(bs, bd), one(fs, fd)])
    dd = ed[:, 1].reshape(3 * EPAD // 128, 128)
    ed = ed.reshape(3, 2, QPR, CH).transpose(0, 2, 1, 3).reshape(3 * QPR, 2, CH)
    return jnp.pad(ed, ((0, EDQ - 3 * QPR), (0, 0), (0, 0))), dd


def kernel(feat_user, feat_item, clicks_src, clicks_dst, buys_src, buys_dst,
           follows_src, follows_dst, W_clicks, W_buys, W_follows,
           W_loop_user, W_loop_item):
    fu = jnp.pad(feat_user, ((0, NPAD - N), (0, 0)))
    fi = jnp.pad(feat_item, ((0, NPAD - N), (0, 0)))
    yc, yb, yf, su, si = _matmuls(fu, fi, W_clicks, W_buys, W_follows,
                                  W_loop_user, W_loop_item)
    ed, dd = _pack_edges(clicks_src, clicks_dst, buys_src, buys_dst,
                         follows_src, follows_dst)
    zrows_c = jnp.zeros((128, D), jnp.float32)
    ones3_c = jnp.broadcast_to(
        (jnp.arange(D)[None, None, :] == jnp.arange(3)[:, None, None])
        .astype(jnp.float32), (3, 128, D))
    s, dg = _sc_scatter(yc, yb, yf, ed, dd, zrows_c, ones3_c)
    s = s.reshape(3, 2, NPAD, D)
    dg = dg.reshape(2, NPAD, D)
    ou, oi = _combine(s, dg, su, si)
    return ou[:N], oi[:N]
